# Initial kernel scaffold; baseline (speedup 1.0000x reference)
import jax
import jax.numpy as jnp
from jax.experimental import pallas as pl


def kernel(x, edge_index, batch, W1, b1, W2, b2, W3, b3, Wl, bl):
    raise NotImplementedError("write your pallas kernel here")


# speedup vs baseline: 7.3017x; 7.3017x over previous
import functools

import jax
import jax.numpy as jnp
from jax import lax
from jax.experimental import pallas as pl
from jax.experimental.pallas import tpu as pltpu
from jax.experimental.pallas import tpu_sc as plsc

F32 = jnp.float32
I32 = jnp.int32

NC, NS = 2, 16
NW = NC * NS
CH = 128
G = 64
BN = 1000
DW = 8


def _sc_mesh():
    return plsc.VectorSubcoreMesh(core_axis_name="c", subcore_axis_name="s",
                                  num_cores=NC, num_subcores=NS)


def _node_part(NH):
    p0 = (NH // NS) // 8 * 8
    tail = NH - NS * p0
    return p0, tail


def _make_deg_kernel(E, N):
    es = E // NS
    full, rem = es // CH, es % CH
    NH = N // NC
    NHP = NH + 8
    p0, tail = _node_part(NH)
    scratch = [
        pltpu.VMEM((1, CH), I32),
        pltpu.VMEM((1, CH), I32),
        pltpu.VMEM((CH, DW), F32),
        pltpu.VMEM((p0, DW), F32),
        pltpu.VMEM_SHARED((NHP, DW), F32),
    ]
    if rem:
        scratch += [pltpu.VMEM((1, rem), I32), pltpu.VMEM((1, rem), I32)]

    @functools.partial(
        pl.kernel,
        out_type=jax.ShapeDtypeStruct((NC, NH, DW), F32),
        mesh=_sc_mesh(),
        scratch_types=scratch,
    )
    def k(dst_hbm, ones_hbm, z_hbm, out_hbm, didx, didx2, ones_v, zb, acc,
          *rest):
        c = lax.axis_index("c")
        s = lax.axis_index("s")
        base = s * es
        lo = c * NH
        pltpu.sync_copy(ones_hbm, ones_v)
        pltpu.sync_copy(z_hbm, zb)
        pltpu.sync_copy(zb, acc.at[pl.ds(s * p0, p0)])
        if tail:
            @pl.when(s == 0)
            def _():
                pltpu.sync_copy(zb.at[pl.ds(0, tail)],
                                acc.at[pl.ds(NS * p0, tail)])
        plsc.subcore_barrier()

        def do_chunk(off, n, dx, dx2):
            pltpu.sync_copy(dst_hbm.at[pl.ds(off, n)], dx.at[0])
            for j in range(n // 16):
                d = dx[0, pl.ds(j * 16, 16)] - lo
                ok = (d >= 0) & (d < NH)
                dx2[0, pl.ds(j * 16, 16)] = jnp.where(ok, d, NH)
            pltpu.sync_copy(ones_v.at[pl.ds(0, n)], acc.at[dx2.at[0]],
                            add=True)

        def body(i, carry):
            do_chunk(base + i * CH, CH, didx, didx2)
            return carry

        lax.fori_loop(0, full, body, 0)
        if rem:
            didx_r, didx2_r = rest
            do_chunk(base + full * CH, rem, didx_r, didx2_r)
        plsc.subcore_barrier()
        pltpu.sync_copy(acc.at[pl.ds(s * p0, p0)], zb)
        pltpu.sync_copy(zb, out_hbm.at[c, pl.ds(s * p0, p0)])
        if tail:
            @pl.when(s == 0)
            def _():
                pltpu.sync_copy(acc.at[pl.ds(NS * p0, tail)],
                                zb.at[pl.ds(0, tail)])
                pltpu.sync_copy(zb.at[pl.ds(0, tail)],
                                out_hbm.at[c, pl.ds(NS * p0, tail)])

    return k


def _make_agg_kernel(E, N, H):
    es = E // NS
    full, rem = es // CH, es % CH
    NH = N // NC
    NHP = NH + 8
    p0, tail = _node_part(NH)
    scratch = [
        pltpu.VMEM((1, CH), I32),
        pltpu.VMEM((1, CH), I32),
        pltpu.VMEM((1, CH), I32),
        pltpu.VMEM((1, CH, H), F32),
        pltpu.VMEM((p0, H), F32),
        pltpu.VMEM_SHARED((NHP, H), F32),
        pltpu.SemaphoreType.DMA,
    ]
    if rem:
        scratch += [pltpu.VMEM((1, rem), I32), pltpu.VMEM((1, rem), I32),
                    pltpu.VMEM((1, rem), I32), pltpu.VMEM((rem, H), F32)]

    @functools.partial(
        pl.kernel,
        out_type=jax.ShapeDtypeStruct((NC, NH, H), F32),
        mesh=_sc_mesh(),
        scratch_types=scratch,
    )
    def k(y_hbm, src_hbm, dst_hbm, z_hbm, out_hbm,
          sidx, didx, didx2, rows, zb, acc, gsem, *rest):
        c = lax.axis_index("c")
        s = lax.axis_index("s")
        base = s * es
        lo = c * NH
        pltpu.sync_copy(z_hbm, zb)
        pltpu.sync_copy(zb, acc.at[pl.ds(s * p0, p0)])
        if tail:
            @pl.when(s == 0)
            def _():
                pltpu.sync_copy(zb.at[pl.ds(0, tail)],
                                acc.at[pl.ds(NS * p0, tail)])
        plsc.subcore_barrier()

        def do_chunk(off, n, sx, dx, dx2, rw):
            pltpu.sync_copy(src_hbm.at[pl.ds(off, n)], sx.at[0])
            pltpu.sync_copy(dst_hbm.at[pl.ds(off, n)], dx.at[0])
            for j in range(n // 16):
                d = dx[0, pl.ds(j * 16, 16)] - lo
                ok = (d >= 0) & (d < NH)
                dx2[0, pl.ds(j * 16, 16)] = jnp.where(ok, d, NH)
            pltpu.async_copy(y_hbm.at[sx.at[0]], rw, gsem).wait()
            pltpu.sync_copy(rw, acc.at[dx2.at[0]], add=True)

        def body(i, carry):
            do_chunk(base + i * CH, CH, sidx, didx, didx2, rows.at[0])
            return carry

        lax.fori_loop(0, full, body, 0)
        if rem:
            sidx_r, didx_r, didx2_r, rows_r = rest
            do_chunk(base + full * CH, rem, sidx_r, didx_r, didx2_r, rows_r)
        plsc.subcore_barrier()
        pltpu.sync_copy(acc.at[pl.ds(s * p0, p0)], zb)
        pltpu.sync_copy(zb, out_hbm.at[c, pl.ds(s * p0, p0)])
        if tail:
            @pl.when(s == 0)
            def _():
                pltpu.sync_copy(acc.at[pl.ds(NS * p0, tail)],
                                zb.at[pl.ds(0, tail)])
                pltpu.sync_copy(zb.at[pl.ds(0, tail)],
                                out_hbm.at[c, pl.ds(NS * p0, tail)])

    return k


def _dinv_block(degp):
    deg = degp[:, 0:1] + 1.0
    return lax.rsqrt(deg)


def _tc_first(x_ref, degp_ref, w_ref, y_ref):
    dinv = _dinv_block(degp_ref[...])
    xw = jnp.dot(x_ref[...], w_ref[...], preferred_element_type=F32)
    y_ref[...] = dinv * xw


def _tc_mid(agg_ref, y_ref, degp_ref, b_ref, w_ref, o_ref):
    dinv = _dinv_block(degp_ref[...])
    h = jnp.maximum(dinv * (agg_ref[...] + y_ref[...]) + b_ref[...], 0.0)
    o_ref[...] = dinv * jnp.dot(h, w_ref[...], preferred_element_type=F32)


def _tc_last(agg_ref, y_ref, degp_ref, b_ref, bat_ref, wl_ref, bl_ref, o_ref):
    n = agg_ref.shape[0]
    dinv = _dinv_block(degp_ref[...])
    h = jnp.maximum(dinv * (agg_ref[...] + y_ref[...]) + b_ref[...], 0.0)
    gid = lax.broadcasted_iota(I32, (G, n), 0)
    mask = (bat_ref[...] == gid).astype(F32)
    counts = jnp.sum(mask, axis=1, keepdims=True)
    pooled = jnp.dot(mask, h, preferred_element_type=F32)
    pooled = pooled / jnp.maximum(counts, 1.0)
    o_ref[...] = jnp.dot(pooled, wl_ref[...],
                         preferred_element_type=F32) + bl_ref[...]


def kernel(x, edge_index, batch, W1, b1, W2, b2, W3, b3, Wl, bl):
    N, D = x.shape
    E = edge_index.shape[1]
    H = W1.shape[1]
    assert E % (NS * 16) == 0 and N % BN == 0 and N % NC == 0

    src = edge_index[0].astype(I32)
    dst = edge_index[1].astype(I32)
    NH = N // NC
    p0, _ = _node_part(NH)

    deg_k = _make_deg_kernel(E, N)
    agg_k = _make_agg_kernel(E, N, H)

    degp = deg_k(dst, jnp.ones((CH, DW), F32),
                 jnp.zeros((p0, DW), F32)).reshape(N, DW)
    zrows = jnp.zeros((p0, H), F32)

    def agg(y):
        return agg_k(y, src, dst, zrows).reshape(N, H)

    grid = (N // BN,)
    full2 = lambda shp: pl.BlockSpec(shp, lambda i: (0, 0))
    row_spec = pl.BlockSpec((BN, H), lambda i: (i, 0))
    degp_spec = pl.BlockSpec((BN, DW), lambda i: (i, 0))

    y1 = pl.pallas_call(
        _tc_first,
        grid=grid,
        in_specs=[pl.BlockSpec((BN, D), lambda i: (i, 0)), degp_spec,
                  full2((D, H))],
        out_specs=row_spec,
        out_shape=jax.ShapeDtypeStruct((N, H), F32),
    )(x, degp, W1)

    mid = pl.pallas_call(
        _tc_mid,
        grid=grid,
        in_specs=[row_spec, row_spec, degp_spec, full2((1, H)),
                  full2((H, H))],
        out_specs=row_spec,
        out_shape=jax.ShapeDtypeStruct((N, H), F32),
    )

    y2 = mid(agg(y1), y1, degp, b1.reshape(1, H), W2)
    y3 = mid(agg(y2), y2, degp, b2.reshape(1, H), W3)
    agg3 = agg(y3)

    out = pl.pallas_call(
        _tc_last,
        in_specs=[pl.BlockSpec((N, H), lambda: (0, 0)),
                  pl.BlockSpec((N, H), lambda: (0, 0)),
                  pl.BlockSpec((N, DW), lambda: (0, 0)),
                  pl.BlockSpec((1, H), lambda: (0, 0)),
                  pl.BlockSpec((1, N), lambda: (0, 0)),
                  pl.BlockSpec((H, 1), lambda: (0, 0)),
                  pl.BlockSpec((1, 1), lambda: (0, 0))],
        out_specs=pl.BlockSpec((G, 1), lambda: (0, 0)),
        out_shape=jax.ShapeDtypeStruct((G, 1), F32),
    )(agg3, y3, degp, b3.reshape(1, H), batch.reshape(1, N).astype(I32),
      Wl, bl.reshape(1, 1))

    return out.reshape(G)


# speedup vs baseline: 11.8281x; 1.6199x over previous
import functools

import jax
import jax.numpy as jnp
from jax import lax
from jax.experimental import pallas as pl
from jax.experimental.pallas import tpu as pltpu
from jax.experimental.pallas import tpu_sc as plsc

F32 = jnp.float32
I32 = jnp.int32

NC, NS = 2, 16
NW = NC * NS
CH = 128
G = 64
BN = 1000
DW = 8
ZRC = 104


def _sc_mesh():
    return plsc.VectorSubcoreMesh(core_axis_name="c", subcore_axis_name="s",
                                  num_cores=NC, num_subcores=NS)


def _node_part(NH):
    p0 = (NH // NS) // 8 * 8
    tail = NH - NS * p0
    return p0, tail


def _make_deg_kernel(E, N):
    es = E // NS
    full, rem = es // CH, es % CH
    NH = N // NC
    NHP = NH + 8
    p0, tail = _node_part(NH)
    scratch = [
        pltpu.VMEM((es,), I32),
        pltpu.VMEM((1, CH), I32),
        pltpu.VMEM((CH, DW), F32),
        pltpu.VMEM((p0, DW), F32),
        pltpu.VMEM_SHARED((NHP, DW), F32),
    ]
    if rem:
        scratch.append(pltpu.VMEM((1, rem), I32))

    @functools.partial(
        pl.kernel,
        out_type=jax.ShapeDtypeStruct((NC, NH, DW), F32),
        mesh=_sc_mesh(),
        scratch_types=scratch,
    )
    def k(dst_hbm, ones_hbm, z_hbm, out_hbm, dall, didx2, ones_v, zb, acc,
          *rest):
        c = lax.axis_index("c")
        s = lax.axis_index("s")
        lo = c * NH
        pltpu.sync_copy(ones_hbm, ones_v)
        pltpu.sync_copy(z_hbm, zb)
        pltpu.sync_copy(zb, acc.at[pl.ds(s * p0, p0)])
        if tail:
            @pl.when(s == 0)
            def _():
                pltpu.sync_copy(zb.at[pl.ds(0, tail)],
                                acc.at[pl.ds(NS * p0, tail)])
        pltpu.sync_copy(dst_hbm.at[pl.ds(s * es, es)], dall)
        plsc.subcore_barrier()

        def do_chunk(kk, n, dx2):
            off = kk * CH
            for j in range(n // 16):
                d = dall[pl.ds(off + j * 16, 16)] - lo
                ok = (d >= 0) & (d < NH)
                dx2[0, pl.ds(j * 16, 16)] = jnp.where(ok, d, NH)
            pltpu.sync_copy(ones_v.at[pl.ds(0, n)], acc.at[dx2.at[0]],
                            add=True)

        def body(i, carry):
            do_chunk(i, CH, didx2)
            return carry

        lax.fori_loop(0, full, body, 0)
        if rem:
            do_chunk(full, rem, rest[0])
        plsc.subcore_barrier()
        pltpu.sync_copy(acc.at[pl.ds(s * p0, p0)], zb)
        pltpu.sync_copy(zb, out_hbm.at[c, pl.ds(s * p0, p0)])
        if tail:
            @pl.when(s == 0)
            def _():
                pltpu.sync_copy(acc.at[pl.ds(NS * p0, tail)],
                                zb.at[pl.ds(0, tail)])
                pltpu.sync_copy(zb.at[pl.ds(0, tail)],
                                out_hbm.at[c, pl.ds(NS * p0, tail)])

    return k


def _make_agg_kernel(E, N, H):
    es = E // NW
    full, rem = es // CH, es % CH
    p0, tail = _node_part(N)
    assert p0 % ZRC == 0 and tail <= ZRC
    scratch = [
        pltpu.VMEM((1, CH), I32),
        pltpu.VMEM((1, CH), I32),
        pltpu.VMEM((1, CH, H), F32),
        pltpu.VMEM((ZRC, H), F32),
        pltpu.VMEM_SHARED((N, H), F32),
        pltpu.SemaphoreType.DMA,
    ]
    if rem:
        scratch += [pltpu.VMEM((1, rem), I32), pltpu.VMEM((1, rem), I32),
                    pltpu.VMEM((rem, H), F32)]

    @functools.partial(
        pl.kernel,
        out_type=jax.ShapeDtypeStruct((NC, N, H), F32),
        mesh=_sc_mesh(),
        scratch_types=scratch,
    )
    def k(y_hbm, src_hbm, dst_hbm, z_hbm, out_hbm,
          sidx, didx, rows, zb, acc, gsem0, *rest):
        c = lax.axis_index("c")
        s = lax.axis_index("s")
        base = (c * NS + s) * es
        pltpu.sync_copy(z_hbm, zb)
        for q in range(p0 // ZRC):
            pltpu.sync_copy(zb, acc.at[pl.ds(s * p0 + q * ZRC, ZRC)])
        if tail:
            @pl.when(s == 0)
            def _():
                pltpu.sync_copy(zb.at[pl.ds(0, tail)],
                                acc.at[pl.ds(NS * p0, tail)])
        plsc.subcore_barrier()

        def body(t, carry):
            off = base + t * CH
            pltpu.sync_copy(src_hbm.at[pl.ds(off, CH)], sidx.at[0])
            pltpu.sync_copy(dst_hbm.at[pl.ds(off, CH)], didx.at[0])
            pltpu.async_copy(y_hbm.at[sidx.at[0]], rows.at[0],
                             gsem0).wait()
            pltpu.sync_copy(rows.at[0], acc.at[didx.at[0]], add=True)
            return carry

        lax.fori_loop(0, full, body, 0)
        if rem:
            sidx_r, didx_r, rows_r = rest
            off = base + full * CH
            pltpu.sync_copy(src_hbm.at[pl.ds(off, rem)], sidx_r.at[0])
            pltpu.sync_copy(dst_hbm.at[pl.ds(off, rem)], didx_r.at[0])
            pltpu.async_copy(y_hbm.at[sidx_r.at[0]], rows_r, gsem0).wait()
            pltpu.sync_copy(rows_r, acc.at[didx_r.at[0]], add=True)
        plsc.subcore_barrier()
        for q in range(p0 // ZRC):
            pltpu.sync_copy(acc.at[pl.ds(s * p0 + q * ZRC, ZRC)], zb)
            pltpu.sync_copy(zb, out_hbm.at[c, pl.ds(s * p0 + q * ZRC, ZRC)])
        if tail:
            @pl.when(s == 0)
            def _():
                pltpu.sync_copy(acc.at[pl.ds(NS * p0, tail)],
                                zb.at[pl.ds(0, tail)])
                pltpu.sync_copy(zb.at[pl.ds(0, tail)],
                                out_hbm.at[c, pl.ds(NS * p0, tail)])

    return k


def _dinv_block(degp):
    deg = degp[:, 0:1] + 1.0
    return lax.rsqrt(deg)


def _tc_first(x_ref, degp_ref, w_ref, y_ref):
    dinv = _dinv_block(degp_ref[...])
    xw = jnp.dot(x_ref[...], w_ref[...], preferred_element_type=F32)
    y_ref[...] = dinv * xw


def _tc_mid(aggp_ref, y_ref, degp_ref, b_ref, w_ref, o_ref):
    dinv = _dinv_block(degp_ref[...])
    agg = aggp_ref[0] + aggp_ref[1]
    h = jnp.maximum(dinv * (agg + y_ref[...]) + b_ref[...], 0.0)
    o_ref[...] = dinv * jnp.dot(h, w_ref[...], preferred_element_type=F32)


def _tc_last(aggp_ref, y_ref, degp_ref, b_ref, bat_ref, wl_ref, bl_ref, o_ref):
    n = aggp_ref.shape[1]
    dinv = _dinv_block(degp_ref[...])
    agg = aggp_ref[0] + aggp_ref[1]
    h = jnp.maximum(dinv * (agg + y_ref[...]) + b_ref[...], 0.0)
    gid = lax.broadcasted_iota(I32, (G, n), 0)
    mask = (bat_ref[...] == gid).astype(F32)
    counts = jnp.sum(mask, axis=1, keepdims=True)
    pooled = jnp.dot(mask, h, preferred_element_type=F32)
    pooled = pooled / jnp.maximum(counts, 1.0)
    o_ref[...] = jnp.dot(pooled, wl_ref[...],
                         preferred_element_type=F32) + bl_ref[...]


def kernel(x, edge_index, batch, W1, b1, W2, b2, W3, b3, Wl, bl):
    N, D = x.shape
    E = edge_index.shape[1]
    H = W1.shape[1]
    assert E % (NS * 16) == 0 and N % BN == 0 and N % NC == 0

    src = edge_index[0].astype(I32)
    dst = edge_index[1].astype(I32)
    p0, _ = _node_part(N)

    deg_k = _make_deg_kernel(E, N)
    agg_k = _make_agg_kernel(E, N, H)

    degp = deg_k(dst, jnp.ones((CH, DW), F32),
                 jnp.zeros((((N // NC) // NS) // 8 * 8, DW),
                           F32)).reshape(N, DW)
    zrows = jnp.zeros((ZRC, H), F32)

    def agg(y):
        return agg_k(y, src, dst, zrows)

    grid = (N // BN,)
    full2 = lambda shp: pl.BlockSpec(shp, lambda i: (0, 0))
    row_spec = pl.BlockSpec((BN, H), lambda i: (i, 0))
    aggp_spec = pl.BlockSpec((NC, BN, H), lambda i: (0, i, 0))
    degp_spec = pl.BlockSpec((BN, DW), lambda i: (i, 0))

    y1 = pl.pallas_call(
        _tc_first,
        grid=grid,
        in_specs=[pl.BlockSpec((BN, D), lambda i: (i, 0)), degp_spec,
                  full2((D, H))],
        out_specs=row_spec,
        out_shape=jax.ShapeDtypeStruct((N, H), F32),
    )(x, degp, W1)

    mid = pl.pallas_call(
        _tc_mid,
        grid=grid,
        in_specs=[aggp_spec, row_spec, degp_spec, full2((1, H)),
                  full2((H, H))],
        out_specs=row_spec,
        out_shape=jax.ShapeDtypeStruct((N, H), F32),
    )

    Ws = jnp.stack([W2, W3, jnp.zeros((H, H), F32)])
    bs = jnp.stack([b1.reshape(1, H), b2.reshape(1, H),
                    jnp.zeros((1, H), F32)])
    nlayers = lax.optimization_barrier(jnp.int32(3))

    def cond(st):
        return st[0] < nlayers

    def body(st):
        i, _, y, _ = st
        aggp = agg(y)
        W = lax.dynamic_index_in_dim(Ws, i, keepdims=False)
        b = lax.dynamic_index_in_dim(bs, i, keepdims=False)
        ynew = mid(aggp, y, degp, b, W)
        return (i + 1, y, ynew, aggp)

    init = (jnp.int32(0), y1, y1, jnp.zeros((NC, N, H), F32))
    _, y3, _, agg3 = lax.while_loop(cond, body, init)

    out = pl.pallas_call(
        _tc_last,
        in_specs=[pl.BlockSpec((NC, N, H), lambda: (0, 0, 0)),
                  pl.BlockSpec((N, H), lambda: (0, 0)),
                  pl.BlockSpec((N, DW), lambda: (0, 0)),
                  pl.BlockSpec((1, H), lambda: (0, 0)),
                  pl.BlockSpec((1, N), lambda: (0, 0)),
                  pl.BlockSpec((H, 1), lambda: (0, 0)),
                  pl.BlockSpec((1, 1), lambda: (0, 0))],
        out_specs=pl.BlockSpec((G, 1), lambda: (0, 0)),
        out_shape=jax.ShapeDtypeStruct((G, 1), F32),
    )(agg3, y3, degp, b3.reshape(1, H), batch.reshape(1, N).astype(I32),
      Wl, bl.reshape(1, 1))

    return out.reshape(G)
